# baseline (speedup 1.0000x reference)
import functools

import jax
import jax.numpy as jnp
from jax import lax
from jax.experimental import pallas as pl
from jax.experimental.pallas import tpu as pltpu
from jax.experimental.pallas import tpu_sc as plsc

N = 10000
E = 160000
NPAD = 10240
EPAD = 163840
NT = 32
RNG = NPAD // NT
EPT = EPAD // NT
CAP = 6016
BR = 40
CB = 512
CBS = 2048
BLK = 256

_DN = lax.GatherDimensionNumbers(
    offset_dims=(), collapsed_slice_dims=(0,), start_index_map=(0,))


def _bcast0(v):
    idx = jnp.zeros((16, 1), jnp.int32)
    return lax.gather(v, idx, _DN, slice_sizes=(1,),
                      mode=lax.GatherScatterMode.PROMISE_IN_BOUNDS)


def _mm_body(H, C, x_ref, w_ref, as_ref, ad_ref, ht_ref, av_ref, bv_ref):
    hb = jnp.dot(x_ref[...], w_ref[...], preferred_element_type=jnp.float32)
    for h in range(H):
        ht_ref[h] = hb[:, h * C:(h + 1) * C].astype(jnp.bfloat16)
    av_ref[...] = jnp.dot(hb, as_ref[...], preferred_element_type=jnp.float32)
    bv_ref[...] = jnp.dot(hb, ad_ref[...], preferred_element_type=jnp.float32)


def _mm(x, w, a_s, a_d, H, C):
    K = x.shape[1]
    M = H * C
    return pl.pallas_call(
        functools.partial(_mm_body, H, C),
        grid=(NPAD // BLK,),
        in_specs=[
            pl.BlockSpec((BLK, K), lambda i: (i, 0)),
            pl.BlockSpec((K, M), lambda i: (0, 0)),
            pl.BlockSpec((M, H), lambda i: (0, 0)),
            pl.BlockSpec((M, H), lambda i: (0, 0)),
        ],
        out_specs=[
            pl.BlockSpec((H, BLK, C), lambda i: (0, i, 0)),
            pl.BlockSpec((BLK, H), lambda i: (i, 0)),
            pl.BlockSpec((BLK, H), lambda i: (i, 0)),
        ],
        out_shape=[
            jax.ShapeDtypeStruct((H, NPAD, C), jnp.bfloat16),
            jax.ShapeDtypeStruct((NPAD, H), jnp.float32),
            jax.ShapeDtypeStruct((NPAD, H), jnp.float32),
        ],
    )(x, w, a_s, a_d)


def _edge_ex_body(H, srcf, dstf, asf, adf, exf, src_v, dst_v, ex_v, as_t, ad_t):
    cc = lax.axis_index("c")
    ss = lax.axis_index("s")
    wid = ss * 2 + cc
    base = wid * EPT
    pltpu.sync_copy(asf, as_t)
    pltpu.sync_copy(adf, ad_t)

    def chunk(ci, _):
        off = base + ci * CB

        pltpu.sync_copy(srcf.at[pl.ds(off, CB)], src_v)
        pltpu.sync_copy(dstf.at[pl.ds(off, CB)], dst_v)

        def vec(k, _):
            s16 = src_v[pl.ds(k * 16, 16)]
            d16 = dst_v[pl.ds(k * 16, 16)]
            for h in range(H):
                fs = s16 * H + h
                fd = d16 * H + h
                a = plsc.load_gather(as_t, [fs // 16, fs % 16])
                b = plsc.load_gather(ad_t, [fd // 16, fd % 16])
                e = a + b
                e = jnp.where(e >= 0.0, e, 0.2 * e)
                ex_v[h, pl.ds(k * 16, 16)] = jnp.exp(e)
            return 0

        lax.fori_loop(0, CB // 16, vec, 0)
        for h in range(H):
            pltpu.sync_copy(ex_v.at[h], exf.at[pl.ds(h * EPAD + off, CB)])
        return 0

    lax.fori_loop(0, EPT // CB, chunk, 0)


def _edge_ex(srcp, dstp, asf, adf, H):
    mesh = plsc.VectorSubcoreMesh(core_axis_name="c", subcore_axis_name="s")
    fn = pl.kernel(
        functools.partial(_edge_ex_body, H),
        out_type=jax.ShapeDtypeStruct((H * EPAD,), jnp.float32),
        mesh=mesh,
        compiler_params=pltpu.CompilerParams(
            needs_layout_passes=False, use_tc_tiling_on_sc=False),
        scratch_types=[
            pltpu.VMEM((CB,), jnp.int32),
            pltpu.VMEM((CB,), jnp.int32),
            pltpu.VMEM((H, CB), jnp.float32),
            pltpu.VMEM((NPAD * H // 16, 16), jnp.float32),
            pltpu.VMEM((NPAD * H // 16, 16), jnp.float32),
        ],
    )
    return fn(srcp, dstp, asf, adf)


def _msg_body(H, dstf, srcf, htf, exf, accf, denf,
              dst_v, src_v, dst_w, src_w, m_ldst, m_src, m_eid, m_ex,
              rows0, rows1, rows2, acc, den, sem, sem0, sem1, sem2):
    cc = lax.axis_index("c")
    ss = lax.axis_index("s")
    wid = ss * 2 + cc
    lo = wid * RNG
    zero16 = jnp.zeros((16,), jnp.float32)
    izero16 = jnp.zeros((16,), jnp.int32)
    lane = lax.iota(jnp.int32, 16)

    def zlists(i, _):
        m_ldst[pl.ds(i * 16, 16)] = izero16
        m_src[pl.ds(i * 16, 16)] = izero16
        m_eid[pl.ds(i * 16, 16)] = izero16
        return 0

    lax.fori_loop(0, CAP // 16, zlists, 0)

    def sdesc(ci, dbuf, sbuf, s):
        off = ci * CBS
        return (pltpu.make_async_copy(dstf.at[pl.ds(off, CBS)], dbuf, s),
                pltpu.make_async_copy(srcf.at[pl.ds(off, CBS)], sbuf, s))

    def scan_work(ci, dbuf, sbuf, cnt):
        off = ci * CBS

        def vec(k, cnt):
            d16 = dbuf[pl.ds(k * 16, 16)]
            m = (d16 >= lo) & (d16 < lo + RNG)
            s16 = sbuf[pl.ds(k * 16, 16)]
            eid16 = off + k * 16 + lane
            w = jnp.minimum(cnt, CAP - 16)
            plsc.store_compressed(m_ldst.at[pl.ds(w, 16)], d16 - lo, mask=m)
            plsc.store_compressed(m_src.at[pl.ds(w, 16)], s16, mask=m)
            plsc.store_compressed(m_eid.at[pl.ds(w, 16)], eid16, mask=m)
            return cnt + jnp.sum(m.astype(jnp.int32))

        return lax.fori_loop(0, CBS // 16, vec, cnt)

    NSC = EPAD // CBS
    for d in sdesc(0, dst_v, src_v, sem0):
        d.start()

    def scan_pair(p, cnt):
        c0 = 2 * p

        @pl.when(c0 + 1 < NSC)
        def _():
            for d in sdesc(c0 + 1, dst_w, src_w, sem1):
                d.start()

        for d in sdesc(c0, dst_v, src_v, sem0):
            d.wait()
        cnt = scan_work(c0, dst_v, src_v, cnt)

        @pl.when(c0 + 2 < NSC)
        def _():
            for d in sdesc(c0 + 2, dst_v, src_v, sem0):
                d.start()

        def odd(cnt):
            for d in sdesc(c0 + 1, dst_w, src_w, sem1):
                d.wait()
            return scan_work(c0 + 1, dst_w, src_w, cnt)

        return lax.cond(c0 + 1 < NSC, odd, lambda c: c, cnt)

    cnt = lax.fori_loop(0, (NSC + 1) // 2, scan_pair, jnp.int32(0))
    nvec = (cnt + 15) // 16
    nex = (cnt + 127) // 128
    nb = (cnt + BR - 1) // BR

    def head_pass(h, _):
        def exg_start(i, _):
            pltpu.make_async_copy(exf.at[m_eid.at[pl.ds(i * 128, 128)]],
                                  m_ex.at[pl.ds(i * 128, 128)], sem).start()
            return 0

        lax.fori_loop(0, nex, exg_start, 0)

        def za(i, _):
            acc[pl.ds(i * 16, 16)] = zero16
            return 0

        lax.fori_loop(0, RNG * 256 // 16, za, 0)

        def zd(i, _):
            den[pl.ds(i * 16, 16)] = zero16
            return 0

        lax.fori_loop(0, RNG // 16, zd, 0)

        def exg_wait(i, _):
            pltpu.make_async_copy(exf.at[m_eid.at[pl.ds(i * 128, 128)]],
                                  m_ex.at[pl.ds(i * 128, 128)], sem).wait()
            return 0

        lax.fori_loop(0, nex, exg_wait, 0)

        def dv(k, _):
            ld16 = m_ldst[pl.ds(k * 16, 16)]
            ex16 = m_ex[pl.ds(k * 16, 16)]
            valid = (k * 16 + lane) < cnt
            plsc.addupdate_scatter(den, [ld16], ex16, mask=valid)
            return 0

        lax.fori_loop(0, nvec, dv, 0)

        def rdesc(b, buf, s):
            return pltpu.make_async_copy(
                htf.at[m_src.at[pl.ds(b * BR, BR)]], buf, s)

        def process(b, buf):
            rb = b * BR
            ne = jnp.minimum(cnt - rb, BR)

            @plsc.parallel_loop(0, jnp.maximum(ne, 0), unroll=2)
            def _(i):
                ab16 = _bcast0(m_ldst[pl.ds(rb + i, 16)] * 256) + lane * 2
                ev = _bcast0(m_ex[pl.ds(rb + i, 16)])
                for j in range(8):
                    pr = buf[i, pl.ds(j * 32, 32)]
                    a, b = plsc.unpack(pr, format=plsc.PackFormat.INTERLEAVED)
                    addr = ab16 + j * 32
                    plsc.addupdate_scatter(acc, [addr], ev * a)
                    plsc.addupdate_scatter(acc, [addr + 1], ev * b)

        bufs = (rows0, rows1, rows2)
        sems = (sem0, sem1, sem2)

        for s in range(2):
            @pl.when(nb > s)
            def _(s=s):
                rdesc(s, bufs[s], sems[s]).start()

        def trip(p, _):
            b0 = 3 * p
            for s in range(3):
                b = b0 + s

                @pl.when(b < nb)
                def _(b=b, s=s):
                    rdesc(b, bufs[s], sems[s]).wait()

                    @pl.when(b + 2 < nb)
                    def _():
                        s2 = (s + 2) % 3
                        rdesc(b + 2, bufs[s2], sems[s2]).start()

                    process(b, bufs[s])

            return 0

        lax.fori_loop(0, (nb + 2) // 3, trip, 0)
        pltpu.sync_copy(acc, accf.at[pl.ds((h * NPAD + lo) * 256, RNG * 256)])
        pltpu.sync_copy(den, denf.at[pl.ds(h * NPAD + lo, RNG)])

        def adv(i, _):
            m_eid[pl.ds(i * 16, 16)] = m_eid[pl.ds(i * 16, 16)] + EPAD
            m_src[pl.ds(i * 16, 16)] = m_src[pl.ds(i * 16, 16)] + NPAD
            return 0

        lax.fori_loop(0, CAP // 16, adv, 0)
        return 0

    lax.fori_loop(0, H, head_pass, 0)


def _msg(dstp, srcp, htf, exf, H):
    mesh = plsc.VectorSubcoreMesh(core_axis_name="c", subcore_axis_name="s")
    fn = pl.kernel(
        functools.partial(_msg_body, H),
        out_type=[
            jax.ShapeDtypeStruct((H * NPAD * 256,), jnp.float32),
            jax.ShapeDtypeStruct((H * NPAD,), jnp.float32),
        ],
        mesh=mesh,
        compiler_params=pltpu.CompilerParams(
            needs_layout_passes=False, use_tc_tiling_on_sc=False),
        scratch_types=[
            pltpu.VMEM((CBS,), jnp.int32),
            pltpu.VMEM((CBS,), jnp.int32),
            pltpu.VMEM((CBS,), jnp.int32),
            pltpu.VMEM((CBS,), jnp.int32),
            pltpu.VMEM((CAP + 16,), jnp.int32),
            pltpu.VMEM((CAP + 16,), jnp.int32),
            pltpu.VMEM((CAP + 16,), jnp.int32),
            pltpu.VMEM((CAP + 16,), jnp.float32),
            pltpu.VMEM((BR, 256), jnp.bfloat16),
            pltpu.VMEM((BR, 256), jnp.bfloat16),
            pltpu.VMEM((BR, 256), jnp.bfloat16),
            pltpu.VMEM((RNG * 256,), jnp.float32),
            pltpu.VMEM((RNG,), jnp.float32),
            pltpu.SemaphoreType.DMA,
            pltpu.SemaphoreType.DMA,
            pltpu.SemaphoreType.DMA,
            pltpu.SemaphoreType.DMA,
        ],
    )
    return fn(dstp, srcp, htf, exf)


def _norm_body(H, C, apply_elu, acc_ref, den_ref, b_ref, o_ref):
    for h in range(H):
        v = acc_ref[h] / (den_ref[h][:, None] + 1e-16) + b_ref[0, h * C:(h + 1) * C]
        if apply_elu:
            v = jnp.where(v > 0.0, v, jnp.exp(jnp.minimum(v, 0.0)) - 1.0)
        o_ref[:, h * C:(h + 1) * C] = v


def _norm(acc, den, b, H, C, apply_elu):
    M = H * C
    return pl.pallas_call(
        functools.partial(_norm_body, H, C, apply_elu),
        grid=(NPAD // BLK,),
        in_specs=[
            pl.BlockSpec((H, BLK, C), lambda i: (0, i, 0)),
            pl.BlockSpec((H, BLK), lambda i: (0, i)),
            pl.BlockSpec((1, M), lambda i: (0, 0)),
        ],
        out_specs=pl.BlockSpec((BLK, M), lambda i: (i, 0)),
        out_shape=jax.ShapeDtypeStruct((NPAD, M), jnp.float32),
    )(acc, den, b)


def _block_diag(a):
    H, C = a.shape
    eye = jnp.eye(H, dtype=a.dtype)
    return (a[:, :, None] * eye[:, None, :]).reshape(H * C, H)


def kernel(x, edge_index, W1, a_src1, a_dst1, b1, W2, a_src2, a_dst2, b2):
    srcp = jnp.concatenate([edge_index[0], jnp.zeros((EPAD - E,), jnp.int32)])
    dstp = jnp.concatenate(
        [edge_index[1], jnp.full((EPAD - E,), NPAD - 1, jnp.int32)])
    xp = jnp.pad(x, ((0, NPAD - N), (0, 0)))

    ht1, as1v, ad1v = _mm(xp, W1, _block_diag(a_src1), _block_diag(a_dst1), 4, 256)
    ex1 = _edge_ex(srcp, dstp, as1v.reshape(-1, 16), ad1v.reshape(-1, 16), 4)
    acc1, den1 = _msg(dstp, srcp, ht1.reshape(4 * NPAD, 256), ex1, 4)
    hin2 = _norm(acc1.reshape(4, NPAD, 256), den1.reshape(4, NPAD),
                 b1.reshape(1, -1), 4, 256, True)

    ht2, as2v, ad2v = _mm(hin2, W2, _block_diag(a_src2), _block_diag(a_dst2), 1, 256)
    ex2 = _edge_ex(srcp, dstp, as2v.reshape(-1, 16), ad2v.reshape(-1, 16), 1)
    acc2, den2 = _msg(dstp, srcp, ht2.reshape(NPAD, 256), ex2, 1)
    out = _norm(acc2.reshape(1, NPAD, 256), den2.reshape(1, NPAD),
                b2.reshape(1, -1), 1, 256, False)
    return out[:N]


# speedup vs baseline: 1.0068x; 1.0068x over previous
import functools

import jax
import jax.numpy as jnp
from jax import lax
from jax.experimental import pallas as pl
from jax.experimental.pallas import tpu as pltpu
from jax.experimental.pallas import tpu_sc as plsc

N = 10000
E = 160000
NPAD = 10240
EPAD = 163840
NT = 32
RNG = NPAD // NT
EPT = EPAD // NT
CAP = 6016
BR = 32
CB = 512
CBS = 2048
BLK = 256

_DN = lax.GatherDimensionNumbers(
    offset_dims=(), collapsed_slice_dims=(0,), start_index_map=(0,))


def _bcast0(v):
    idx = jnp.zeros((16, 1), jnp.int32)
    return lax.gather(v, idx, _DN, slice_sizes=(1,),
                      mode=lax.GatherScatterMode.PROMISE_IN_BOUNDS)


def _mm_body(H, C, x_ref, w_ref, as_ref, ad_ref, ht_ref, av_ref, bv_ref):
    hb = jnp.dot(x_ref[...], w_ref[...], preferred_element_type=jnp.float32)
    for h in range(H):
        ht_ref[h] = hb[:, h * C:(h + 1) * C].astype(jnp.bfloat16)
    av_ref[...] = jnp.dot(hb, as_ref[...], preferred_element_type=jnp.float32)
    bv_ref[...] = jnp.dot(hb, ad_ref[...], preferred_element_type=jnp.float32)


def _mm(x, w, a_s, a_d, H, C):
    K = x.shape[1]
    M = H * C
    return pl.pallas_call(
        functools.partial(_mm_body, H, C),
        grid=(NPAD // BLK,),
        in_specs=[
            pl.BlockSpec((BLK, K), lambda i: (i, 0)),
            pl.BlockSpec((K, M), lambda i: (0, 0)),
            pl.BlockSpec((M, H), lambda i: (0, 0)),
            pl.BlockSpec((M, H), lambda i: (0, 0)),
        ],
        out_specs=[
            pl.BlockSpec((H, BLK, C), lambda i: (0, i, 0)),
            pl.BlockSpec((BLK, H), lambda i: (i, 0)),
            pl.BlockSpec((BLK, H), lambda i: (i, 0)),
        ],
        out_shape=[
            jax.ShapeDtypeStruct((H, NPAD, C), jnp.bfloat16),
            jax.ShapeDtypeStruct((NPAD, H), jnp.float32),
            jax.ShapeDtypeStruct((NPAD, H), jnp.float32),
        ],
    )(x, w, a_s, a_d)


def _edge_ex_body(H, srcf, dstf, asf, adf, exf, src_v, dst_v, ex_v, as_t, ad_t):
    cc = lax.axis_index("c")
    ss = lax.axis_index("s")
    wid = ss * 2 + cc
    base = wid * EPT
    pltpu.sync_copy(asf, as_t)
    pltpu.sync_copy(adf, ad_t)

    def chunk(ci, _):
        off = base + ci * CB

        pltpu.sync_copy(srcf.at[pl.ds(off, CB)], src_v)
        pltpu.sync_copy(dstf.at[pl.ds(off, CB)], dst_v)

        def vec(k, _):
            s16 = src_v[pl.ds(k * 16, 16)]
            d16 = dst_v[pl.ds(k * 16, 16)]
            for h in range(H):
                fs = s16 * H + h
                fd = d16 * H + h
                a = plsc.load_gather(as_t, [fs // 16, fs % 16])
                b = plsc.load_gather(ad_t, [fd // 16, fd % 16])
                e = a + b
                e = jnp.where(e >= 0.0, e, 0.2 * e)
                ex_v[h, pl.ds(k * 16, 16)] = jnp.exp(e)
            return 0

        lax.fori_loop(0, CB // 16, vec, 0)
        for h in range(H):
            pltpu.sync_copy(ex_v.at[h], exf.at[pl.ds(h * EPAD + off, CB)])
        return 0

    lax.fori_loop(0, EPT // CB, chunk, 0)


def _edge_ex(srcp, dstp, asf, adf, H):
    mesh = plsc.VectorSubcoreMesh(core_axis_name="c", subcore_axis_name="s")
    fn = pl.kernel(
        functools.partial(_edge_ex_body, H),
        out_type=jax.ShapeDtypeStruct((H * EPAD,), jnp.float32),
        mesh=mesh,
        compiler_params=pltpu.CompilerParams(
            needs_layout_passes=False, use_tc_tiling_on_sc=False),
        scratch_types=[
            pltpu.VMEM((CB,), jnp.int32),
            pltpu.VMEM((CB,), jnp.int32),
            pltpu.VMEM((H, CB), jnp.float32),
            pltpu.VMEM((NPAD * H // 16, 16), jnp.float32),
            pltpu.VMEM((NPAD * H // 16, 16), jnp.float32),
        ],
    )
    return fn(srcp, dstp, asf, adf)


def _msg_body(H, dstf, srcf, htf, exf, accf, denf,
              dst_v, src_v, dst_w, src_w, m_ldst, m_src, m_eid, m_ex,
              rows0, rows1, rows2, acc, den, sem, sem0, sem1, sem2):
    cc = lax.axis_index("c")
    ss = lax.axis_index("s")
    wid = ss * 2 + cc
    lo = wid * RNG
    zero16 = jnp.zeros((16,), jnp.float32)
    izero16 = jnp.zeros((16,), jnp.int32)
    lane = lax.iota(jnp.int32, 16)

    def zlists(i, _):
        m_ldst[pl.ds(i * 16, 16)] = izero16
        m_src[pl.ds(i * 16, 16)] = izero16
        m_eid[pl.ds(i * 16, 16)] = izero16
        return 0

    lax.fori_loop(0, CAP // 16, zlists, 0)

    def sdesc(ci, dbuf, sbuf, s):
        off = ci * CBS
        return (pltpu.make_async_copy(dstf.at[pl.ds(off, CBS)], dbuf, s),
                pltpu.make_async_copy(srcf.at[pl.ds(off, CBS)], sbuf, s))

    def scan_work(ci, dbuf, sbuf, cnt):
        off = ci * CBS

        def vec(k, cnt):
            d16 = dbuf[pl.ds(k * 16, 16)]
            m = (d16 >= lo) & (d16 < lo + RNG)
            s16 = sbuf[pl.ds(k * 16, 16)]
            eid16 = off + k * 16 + lane
            w = jnp.minimum(cnt, CAP - 16)
            plsc.store_compressed(m_ldst.at[pl.ds(w, 16)], d16 - lo, mask=m)
            plsc.store_compressed(m_src.at[pl.ds(w, 16)], s16, mask=m)
            plsc.store_compressed(m_eid.at[pl.ds(w, 16)], eid16, mask=m)
            return cnt + jnp.sum(m.astype(jnp.int32))

        return lax.fori_loop(0, CBS // 16, vec, cnt)

    NSC = EPAD // CBS
    for d in sdesc(0, dst_v, src_v, sem0):
        d.start()

    def scan_pair(p, cnt):
        c0 = 2 * p

        @pl.when(c0 + 1 < NSC)
        def _():
            for d in sdesc(c0 + 1, dst_w, src_w, sem1):
                d.start()

        for d in sdesc(c0, dst_v, src_v, sem0):
            d.wait()
        cnt = scan_work(c0, dst_v, src_v, cnt)

        @pl.when(c0 + 2 < NSC)
        def _():
            for d in sdesc(c0 + 2, dst_v, src_v, sem0):
                d.start()

        def odd(cnt):
            for d in sdesc(c0 + 1, dst_w, src_w, sem1):
                d.wait()
            return scan_work(c0 + 1, dst_w, src_w, cnt)

        return lax.cond(c0 + 1 < NSC, odd, lambda c: c, cnt)

    cnt = lax.fori_loop(0, (NSC + 1) // 2, scan_pair, jnp.int32(0))
    nvec = (cnt + 15) // 16
    nex = (cnt + 127) // 128
    nb = (cnt + BR - 1) // BR

    def head_pass(h, _):
        def exg_start(i, _):
            pltpu.make_async_copy(exf.at[m_eid.at[pl.ds(i * 128, 128)]],
                                  m_ex.at[pl.ds(i * 128, 128)], sem).start()
            return 0

        lax.fori_loop(0, nex, exg_start, 0)

        def za(i, _):
            acc[pl.ds(i * 16, 16)] = zero16
            return 0

        lax.fori_loop(0, RNG * 256 // 16, za, 0)

        def zd(i, _):
            den[pl.ds(i * 16, 16)] = zero16
            return 0

        lax.fori_loop(0, RNG // 16, zd, 0)

        def exg_wait(i, _):
            pltpu.make_async_copy(exf.at[m_eid.at[pl.ds(i * 128, 128)]],
                                  m_ex.at[pl.ds(i * 128, 128)], sem).wait()
            return 0

        lax.fori_loop(0, nex, exg_wait, 0)

        def dv(k, _):
            ld16 = m_ldst[pl.ds(k * 16, 16)]
            ex16 = m_ex[pl.ds(k * 16, 16)]
            valid = (k * 16 + lane) < cnt
            plsc.addupdate_scatter(den, [ld16], ex16, mask=valid)
            return 0

        lax.fori_loop(0, nvec, dv, 0)

        def rdesc(b, buf, s):
            return pltpu.make_async_copy(
                htf.at[m_src.at[pl.ds(b * BR, BR)]], buf, s)

        def process(b, buf):
            rb = b * BR
            ne = jnp.minimum(cnt - rb, BR)

            @plsc.parallel_loop(0, jnp.maximum(ne, 0), unroll=2)
            def _(i):
                ab16 = _bcast0(m_ldst[pl.ds(rb + i, 16)] * 256) + lane * 2
                ev = _bcast0(m_ex[pl.ds(rb + i, 16)])
                for j in range(8):
                    pr = buf[i, pl.ds(j * 32, 32)]
                    a, b = plsc.unpack(pr, format=plsc.PackFormat.INTERLEAVED)
                    addr = ab16 + j * 32
                    plsc.addupdate_scatter(acc, [addr], ev * a)
                    plsc.addupdate_scatter(acc, [addr + 1], ev * b)

        bufs = (rows0, rows1, rows2)
        sems = (sem0, sem1, sem2)

        for s in range(2):
            @pl.when(nb > s)
            def _(s=s):
                rdesc(s, bufs[s], sems[s]).start()

        def trip(p, _):
            b0 = 3 * p
            for s in range(3):
                b = b0 + s

                @pl.when(b < nb)
                def _(b=b, s=s):
                    rdesc(b, bufs[s], sems[s]).wait()

                    @pl.when(b + 2 < nb)
                    def _():
                        s2 = (s + 2) % 3
                        rdesc(b + 2, bufs[s2], sems[s2]).start()

                    process(b, bufs[s])

            return 0

        lax.fori_loop(0, (nb + 2) // 3, trip, 0)
        pltpu.sync_copy(acc, accf.at[pl.ds((h * NPAD + lo) * 256, RNG * 256)])
        pltpu.sync_copy(den, denf.at[pl.ds(h * NPAD + lo, RNG)])

        def adv(i, _):
            m_eid[pl.ds(i * 16, 16)] = m_eid[pl.ds(i * 16, 16)] + EPAD
            m_src[pl.ds(i * 16, 16)] = m_src[pl.ds(i * 16, 16)] + NPAD
            return 0

        lax.fori_loop(0, CAP // 16, adv, 0)
        return 0

    lax.fori_loop(0, H, head_pass, 0)


def _msg(dstp, srcp, htf, exf, H):
    mesh = plsc.VectorSubcoreMesh(core_axis_name="c", subcore_axis_name="s")
    fn = pl.kernel(
        functools.partial(_msg_body, H),
        out_type=[
            jax.ShapeDtypeStruct((H * NPAD * 256,), jnp.float32),
            jax.ShapeDtypeStruct((H * NPAD,), jnp.float32),
        ],
        mesh=mesh,
        compiler_params=pltpu.CompilerParams(
            needs_layout_passes=False, use_tc_tiling_on_sc=False),
        scratch_types=[
            pltpu.VMEM((CBS,), jnp.int32),
            pltpu.VMEM((CBS,), jnp.int32),
            pltpu.VMEM((CBS,), jnp.int32),
            pltpu.VMEM((CBS,), jnp.int32),
            pltpu.VMEM((CAP + 16,), jnp.int32),
            pltpu.VMEM((CAP + 16,), jnp.int32),
            pltpu.VMEM((CAP + 16,), jnp.int32),
            pltpu.VMEM((CAP + 16,), jnp.float32),
            pltpu.VMEM((BR, 256), jnp.bfloat16),
            pltpu.VMEM((BR, 256), jnp.bfloat16),
            pltpu.VMEM((BR, 256), jnp.bfloat16),
            pltpu.VMEM((RNG * 256,), jnp.float32),
            pltpu.VMEM((RNG,), jnp.float32),
            pltpu.SemaphoreType.DMA,
            pltpu.SemaphoreType.DMA,
            pltpu.SemaphoreType.DMA,
            pltpu.SemaphoreType.DMA,
        ],
    )
    return fn(dstp, srcp, htf, exf)


def _norm_body(H, C, apply_elu, acc_ref, den_ref, b_ref, o_ref):
    for h in range(H):
        v = acc_ref[h] / (den_ref[h][:, None] + 1e-16) + b_ref[0, h * C:(h + 1) * C]
        if apply_elu:
            v = jnp.where(v > 0.0, v, jnp.exp(jnp.minimum(v, 0.0)) - 1.0)
        o_ref[:, h * C:(h + 1) * C] = v


def _norm(acc, den, b, H, C, apply_elu):
    M = H * C
    return pl.pallas_call(
        functools.partial(_norm_body, H, C, apply_elu),
        grid=(NPAD // BLK,),
        in_specs=[
            pl.BlockSpec((H, BLK, C), lambda i: (0, i, 0)),
            pl.BlockSpec((H, BLK), lambda i: (0, i)),
            pl.BlockSpec((1, M), lambda i: (0, 0)),
        ],
        out_specs=pl.BlockSpec((BLK, M), lambda i: (i, 0)),
        out_shape=jax.ShapeDtypeStruct((NPAD, M), jnp.float32),
    )(acc, den, b)


def _block_diag(a):
    H, C = a.shape
    eye = jnp.eye(H, dtype=a.dtype)
    return (a[:, :, None] * eye[:, None, :]).reshape(H * C, H)


def kernel(x, edge_index, W1, a_src1, a_dst1, b1, W2, a_src2, a_dst2, b2):
    srcp = jnp.concatenate([edge_index[0], jnp.zeros((EPAD - E,), jnp.int32)])
    dstp = jnp.concatenate(
        [edge_index[1], jnp.full((EPAD - E,), NPAD - 1, jnp.int32)])
    xp = jnp.pad(x, ((0, NPAD - N), (0, 0)))

    ht1, as1v, ad1v = _mm(xp, W1, _block_diag(a_src1), _block_diag(a_dst1), 4, 256)
    ex1 = _edge_ex(srcp, dstp, as1v.reshape(-1, 16), ad1v.reshape(-1, 16), 4)
    acc1, den1 = _msg(dstp, srcp, ht1.reshape(4 * NPAD, 256), ex1, 4)
    hin2 = _norm(acc1.reshape(4, NPAD, 256), den1.reshape(4, NPAD),
                 b1.reshape(1, -1), 4, 256, True)

    ht2, as2v, ad2v = _mm(hin2, W2, _block_diag(a_src2), _block_diag(a_dst2), 1, 256)
    ex2 = _edge_ex(srcp, dstp, as2v.reshape(-1, 16), ad2v.reshape(-1, 16), 1)
    acc2, den2 = _msg(dstp, srcp, ht2.reshape(NPAD, 256), ex2, 1)
    out = _norm(acc2.reshape(1, NPAD, 256), den2.reshape(1, NPAD),
                b2.reshape(1, -1), 1, 256, False)
    return out[:N]
